# Initial kernel scaffold; baseline (speedup 1.0000x reference)
import functools

import jax
import jax.numpy as jnp
from jax import lax
from jax.experimental import pallas as pl
from jax.experimental.pallas import tpu as pltpu
from jax.experimental.pallas import tpu_sc as plsc

N = 10000
E = 320000
D = 128

NC = 2
NS = 16
NW = NC * NS

CH = 128
EPW = 10112
E_PAD = NW * EPW
NCHUNK = EPW // CH

NACC = 10240
RPT = NACC // NS

RB = 1024
NB = NACC // RB
GRID = (N + RB - 1) // RB


_MESH = plsc.VectorSubcoreMesh(core_axis_name="c", subcore_axis_name="s")


@functools.partial(
    pl.kernel,
    out_type=jax.ShapeDtypeStruct((NC * NACC, 8), jnp.float32),
    mesh=_MESH,
    scratch_types=[
        pltpu.VMEM((CH,), jnp.int32),
        pltpu.VMEM((CH, 8), jnp.float32),
        pltpu.VMEM_SHARED((NACC, 8), jnp.float32),
    ],
)
def _sc_deg(col_hbm, zeros_hbm, ones_hbm, out_hbm, colidx_v, ones_v, acc_sh):
    c = lax.axis_index("c")
    s = lax.axis_index("s")
    pltpu.sync_copy(zeros_hbm, acc_sh.at[pl.ds(s * RPT, RPT)])
    pltpu.sync_copy(ones_hbm, ones_v)
    plsc.subcore_barrier()
    ebase = (c * NS + s) * EPW

    def body(j, carry):
        pltpu.sync_copy(col_hbm.at[pl.ds(ebase + j * CH, CH)], colidx_v)
        pltpu.sync_copy(ones_v, acc_sh.at[colidx_v], add=True)
        return carry

    lax.fori_loop(0, NCHUNK, body, 0)
    plsc.subcore_barrier()
    pltpu.sync_copy(
        acc_sh.at[pl.ds(s * RPT, RPT)],
        out_hbm.at[pl.ds(c * NACC + s * RPT, RPT)],
    )


@functools.partial(
    pl.kernel,
    out_type=jax.ShapeDtypeStruct((NC * NACC, D), jnp.float32),
    mesh=_MESH,
    scratch_types=[
        pltpu.VMEM((CH,), jnp.int32),
        pltpu.VMEM((CH,), jnp.int32),
        pltpu.VMEM((CH, D), jnp.float32),
        pltpu.VMEM_SHARED((NACC, D), jnp.float32),
        pltpu.SemaphoreType.DMA,
    ],
)
def _sc_agg(g_hbm, row_hbm, col_hbm, zeros_hbm, out_hbm,
            rowidx_v, colidx_v, rows_v, acc_sh, sem):
    c = lax.axis_index("c")
    s = lax.axis_index("s")
    pltpu.sync_copy(zeros_hbm, acc_sh.at[pl.ds(s * RPT, RPT)])
    plsc.subcore_barrier()
    ebase = (c * NS + s) * EPW

    def body(j, carry):
        base = ebase + j * CH
        pltpu.sync_copy(row_hbm.at[pl.ds(base, CH)], rowidx_v)
        pltpu.sync_copy(col_hbm.at[pl.ds(base, CH)], colidx_v)
        pltpu.async_copy(g_hbm.at[rowidx_v], rows_v, sem).wait()
        pltpu.sync_copy(rows_v, acc_sh.at[colidx_v], add=True)
        return carry

    lax.fori_loop(0, NCHUNK, body, 0)
    plsc.subcore_barrier()
    pltpu.sync_copy(
        acc_sh.at[pl.ds(s * RPT, RPT)],
        out_hbm.at[pl.ds(c * NACC + s * RPT, RPT)],
    )


def _k1_body(dA, dB, x_ref, w_ref, g_ref, dis_ref):
    deg = dA[:, 0:1] + dB[:, 0:1] + 1.0
    dis = lax.rsqrt(deg)
    g_ref[...] = jnp.dot(x_ref[...], w_ref[...],
                         preferred_element_type=jnp.float32) * dis
    dis_ref[...] = dis


def _tc_k1(degp, x, W1):
    return pl.pallas_call(
        _k1_body,
        grid=(GRID,),
        in_specs=[
            pl.BlockSpec((RB, 8), lambda i: (i, 0)),
            pl.BlockSpec((RB, 8), lambda i: (i + NB, 0)),
            pl.BlockSpec((RB, D), lambda i: (i, 0)),
            pl.BlockSpec((D, D), lambda i: (0, 0)),
        ],
        out_specs=[
            pl.BlockSpec((RB, D), lambda i: (i, 0)),
            pl.BlockSpec((RB, 1), lambda i: (i, 0)),
        ],
        out_shape=[
            jax.ShapeDtypeStruct((N, D), jnp.float32),
            jax.ShapeDtypeStruct((N, 1), jnp.float32),
        ],
    )(degp, degp, x, W1)


def _k2_body(aA, aB, g1_ref, dis_ref, b1_ref, w_ref, g2_ref):
    dis = dis_ref[...]
    h = dis * (aA[...] + aB[...] + g1_ref[...]) + b1_ref[...][None, :]
    h = jnp.maximum(h, 0.0)
    g2_ref[...] = jnp.dot(h, w_ref[...],
                          preferred_element_type=jnp.float32) * dis


def _tc_k2(acc1, g1, dis, b1, W2):
    return pl.pallas_call(
        _k2_body,
        grid=(GRID,),
        in_specs=[
            pl.BlockSpec((RB, D), lambda i: (i, 0)),
            pl.BlockSpec((RB, D), lambda i: (i + NB, 0)),
            pl.BlockSpec((RB, D), lambda i: (i, 0)),
            pl.BlockSpec((RB, 1), lambda i: (i, 0)),
            pl.BlockSpec((D,), lambda i: (0,)),
            pl.BlockSpec((D, D), lambda i: (0, 0)),
        ],
        out_specs=pl.BlockSpec((RB, D), lambda i: (i, 0)),
        out_shape=jax.ShapeDtypeStruct((N, D), jnp.float32),
    )(acc1, acc1, g1, dis, b1, W2)


def _k3_body(aA, aB, g2_ref, dis_ref, b2_ref, out_ref):
    out_ref[...] = (dis_ref[...] * (aA[...] + aB[...] + g2_ref[...])
                    + b2_ref[...][None, :])


def _tc_k3(acc2, g2, dis, b2):
    return pl.pallas_call(
        _k3_body,
        grid=(GRID,),
        in_specs=[
            pl.BlockSpec((RB, D), lambda i: (i, 0)),
            pl.BlockSpec((RB, D), lambda i: (i + NB, 0)),
            pl.BlockSpec((RB, D), lambda i: (i, 0)),
            pl.BlockSpec((RB, 1), lambda i: (i, 0)),
            pl.BlockSpec((D,), lambda i: (0,)),
        ],
        out_specs=pl.BlockSpec((RB, D), lambda i: (i, 0)),
        out_shape=jax.ShapeDtypeStruct((N, D), jnp.float32),
    )(acc2, acc2, g2, dis, b2)


def kernel(x, edge_index, W1, b1, W2, b2):
    row = edge_index[0].astype(jnp.int32)
    col = edge_index[1].astype(jnp.int32)
    pad = E_PAD - E
    row_p = jnp.concatenate([row, jnp.zeros((pad,), jnp.int32)])
    col_p = jnp.concatenate([col, jnp.full((pad,), N, jnp.int32)])

    zeros_deg = jnp.zeros((RPT, 8), jnp.float32)
    ones8 = jnp.ones((CH, 8), jnp.float32)
    zeros_rows = jnp.zeros((RPT, D), jnp.float32)

    degp = _sc_deg(col_p, zeros_deg, ones8)
    g1, dis = _tc_k1(degp, x, W1)
    acc1 = _sc_agg(g1, row_p, col_p, zeros_rows)
    g2 = _tc_k2(acc1, g1, dis, b1, W2)
    acc2 = _sc_agg(g2, row_p, col_p, zeros_rows)
    return _tc_k3(acc2, g2, dis, b2)


# speedup vs baseline: 10.5415x; 10.5415x over previous
import functools

import jax
import jax.numpy as jnp
from jax import lax
from jax.experimental import pallas as pl
from jax.experimental.pallas import tpu as pltpu
from jax.experimental.pallas import tpu_sc as plsc

N = 10000
E = 320000
D = 128

NC = 2
NS = 16
NW = NC * NS

CH = 128
EPW = 10112
E_PAD = NW * EPW
NCHUNK = EPW // CH

NACC = 10240
RPT = NACC // NS

RB = 1024
NB = NACC // RB
GRID = (N + RB - 1) // RB


@functools.cache
def _sc_deg_kernel():
    mesh = plsc.VectorSubcoreMesh(core_axis_name="c", subcore_axis_name="s")

    @functools.partial(
        pl.kernel,
        out_type=jax.ShapeDtypeStruct((NC * NACC, D), jnp.float32),
        mesh=mesh,
        scratch_types=[
            pltpu.VMEM((CH,), jnp.int32),
            pltpu.VMEM((CH, D), jnp.float32),
            pltpu.VMEM_SHARED((NACC, D), jnp.float32),
        ],
    )
    def _sc_deg(col_hbm, zeros_hbm, ones_hbm, out_hbm, colidx_v, ones_v, acc_sh):
        c = lax.axis_index("c")
        s = lax.axis_index("s")
        pltpu.sync_copy(zeros_hbm, acc_sh.at[pl.ds(s * RPT, RPT)])
        pltpu.sync_copy(ones_hbm, ones_v)
        plsc.subcore_barrier()
        ebase = (c * NS + s) * EPW

        def body(j, carry):
            pltpu.sync_copy(col_hbm.at[pl.ds(ebase + j * CH, CH)], colidx_v)
            pltpu.sync_copy(ones_v, acc_sh.at[colidx_v], add=True)
            return carry

        lax.fori_loop(0, NCHUNK, body, 0)
        plsc.subcore_barrier()
        pltpu.sync_copy(
            acc_sh.at[pl.ds(s * RPT, RPT)],
            out_hbm.at[pl.ds(c * NACC + s * RPT, RPT)],
        )

    return _sc_deg


@functools.cache
def _sc_agg_kernel():
    mesh = plsc.VectorSubcoreMesh(core_axis_name="c", subcore_axis_name="s")

    @functools.partial(
        pl.kernel,
        out_type=jax.ShapeDtypeStruct((NC * NACC, D), jnp.float32),
        mesh=mesh,
        scratch_types=[
            pltpu.VMEM((CH,), jnp.int32),
            pltpu.VMEM((CH,), jnp.int32),
            pltpu.VMEM((CH, D), jnp.float32),
            pltpu.VMEM_SHARED((NACC, D), jnp.float32),
            pltpu.SemaphoreType.DMA,
        ],
    )
    def _sc_agg(g_hbm, row_hbm, col_hbm, zeros_hbm, out_hbm,
                rowidx_v, colidx_v, rows_v, acc_sh, sem):
        c = lax.axis_index("c")
        s = lax.axis_index("s")
        pltpu.sync_copy(zeros_hbm, acc_sh.at[pl.ds(s * RPT, RPT)])
        plsc.subcore_barrier()
        ebase = (c * NS + s) * EPW

        def body(j, carry):
            base = ebase + j * CH
            pltpu.sync_copy(row_hbm.at[pl.ds(base, CH)], rowidx_v)
            pltpu.sync_copy(col_hbm.at[pl.ds(base, CH)], colidx_v)
            pltpu.async_copy(g_hbm.at[rowidx_v], rows_v, sem).wait()
            pltpu.sync_copy(rows_v, acc_sh.at[colidx_v], add=True)
            return carry

        lax.fori_loop(0, NCHUNK, body, 0)
        plsc.subcore_barrier()
        pltpu.sync_copy(
            acc_sh.at[pl.ds(s * RPT, RPT)],
            out_hbm.at[pl.ds(c * NACC + s * RPT, RPT)],
        )

    return _sc_agg


def _k1_body(dA, dB, x_ref, w_ref, g_ref, dis_ref):
    deg = dA[:, 0:1] + dB[:, 0:1] + 1.0
    dis = lax.rsqrt(deg)
    g_ref[...] = jnp.dot(x_ref[...], w_ref[...],
                         preferred_element_type=jnp.float32) * dis
    dis_ref[...] = dis


def _tc_k1(degp, x, W1):
    return pl.pallas_call(
        _k1_body,
        grid=(GRID,),
        in_specs=[
            pl.BlockSpec((RB, D), lambda i: (i, 0)),
            pl.BlockSpec((RB, D), lambda i: (i + NB, 0)),
            pl.BlockSpec((RB, D), lambda i: (i, 0)),
            pl.BlockSpec((D, D), lambda i: (0, 0)),
        ],
        out_specs=[
            pl.BlockSpec((RB, D), lambda i: (i, 0)),
            pl.BlockSpec((RB, 1), lambda i: (i, 0)),
        ],
        out_shape=[
            jax.ShapeDtypeStruct((N, D), jnp.float32),
            jax.ShapeDtypeStruct((N, 1), jnp.float32),
        ],
    )(degp, degp, x, W1)


def _k2_body(aA, aB, g1_ref, dis_ref, b1_ref, w_ref, g2_ref):
    dis = dis_ref[...]
    h = dis * (aA[...] + aB[...] + g1_ref[...]) + b1_ref[...][None, :]
    h = jnp.maximum(h, 0.0)
    g2_ref[...] = jnp.dot(h, w_ref[...],
                          preferred_element_type=jnp.float32) * dis


def _tc_k2(acc1, g1, dis, b1, W2):
    return pl.pallas_call(
        _k2_body,
        grid=(GRID,),
        in_specs=[
            pl.BlockSpec((RB, D), lambda i: (i, 0)),
            pl.BlockSpec((RB, D), lambda i: (i + NB, 0)),
            pl.BlockSpec((RB, D), lambda i: (i, 0)),
            pl.BlockSpec((RB, 1), lambda i: (i, 0)),
            pl.BlockSpec((D,), lambda i: (0,)),
            pl.BlockSpec((D, D), lambda i: (0, 0)),
        ],
        out_specs=pl.BlockSpec((RB, D), lambda i: (i, 0)),
        out_shape=jax.ShapeDtypeStruct((N, D), jnp.float32),
    )(acc1, acc1, g1, dis, b1, W2)


def _k3_body(aA, aB, g2_ref, dis_ref, b2_ref, out_ref):
    out_ref[...] = (dis_ref[...] * (aA[...] + aB[...] + g2_ref[...])
                    + b2_ref[...][None, :])


def _tc_k3(acc2, g2, dis, b2):
    return pl.pallas_call(
        _k3_body,
        grid=(GRID,),
        in_specs=[
            pl.BlockSpec((RB, D), lambda i: (i, 0)),
            pl.BlockSpec((RB, D), lambda i: (i + NB, 0)),
            pl.BlockSpec((RB, D), lambda i: (i, 0)),
            pl.BlockSpec((RB, 1), lambda i: (i, 0)),
            pl.BlockSpec((D,), lambda i: (0,)),
        ],
        out_specs=pl.BlockSpec((RB, D), lambda i: (i, 0)),
        out_shape=jax.ShapeDtypeStruct((N, D), jnp.float32),
    )(acc2, acc2, g2, dis, b2)


def kernel(x, edge_index, W1, b1, W2, b2):
    row = edge_index[0].astype(jnp.int32)
    col = edge_index[1].astype(jnp.int32)
    pad = E_PAD - E
    row_p = jnp.concatenate([row, jnp.zeros((pad,), jnp.int32)])
    col_p = jnp.concatenate([col, jnp.full((pad,), N, jnp.int32)])

    ones_rows = jnp.ones((CH, D), jnp.float32)
    zeros_rows = jnp.zeros((RPT, D), jnp.float32)

    degp = _sc_deg_kernel()(col_p, zeros_rows, ones_rows)
    g1, dis = _tc_k1(degp, x, W1)
    acc1 = _sc_agg_kernel()(g1, row_p, col_p, zeros_rows)
    g2 = _tc_k2(acc1, g1, dis, b1, W2)
    acc2 = _sc_agg_kernel()(g2, row_p, col_p, zeros_rows)
    return _tc_k3(acc2, g2, dis, b2)
